# baseline (speedup 1.0000x reference)
import functools

import jax
import jax.numpy as jnp
from jax import lax
from jax.experimental import pallas as pl
from jax.experimental.pallas import tpu as pltpu
from jax.experimental.pallas import tpu_sc as plsc

NPIN = 3200000
NNET = 100000
NSUB = 32
NPN = 3200
NPAD = NSUB * NPN
CHUNK = 8192
PAD = 3 * CHUNK
MASK19 = (1 << 19) - 1
SENT = 1 << 20


def _vshift(v, idx):
    return lax.gather(
        v, idx[:, None],
        lax.GatherDimensionNumbers(offset_dims=(), collapsed_slice_dims=(0,),
                                   start_index_map=(0,)),
        (1,), mode=lax.GatherScatterMode.PROMISE_IN_BOUNDS)


_mesh = plsc.VectorSubcoreMesh(core_axis_name="c", subcore_axis_name="s")


@functools.partial(
    pl.kernel,
    out_type=jax.ShapeDtypeStruct((8 * NPAD,), jnp.float32),
    mesh=_mesh,
    compiler_params=pltpu.CompilerParams(needs_layout_passes=False),
    scratch_types=[
        pltpu.VMEM((CHUNK,), jnp.float32),
        pltpu.VMEM((CHUNK,), jnp.float32),
        pltpu.VMEM((CHUNK,), jnp.float32),
        pltpu.VMEM((CHUNK,), jnp.float32),
        pltpu.VMEM((CHUNK,), jnp.int32),
        pltpu.VMEM((CHUNK,), jnp.int32),
        pltpu.SemaphoreType.DMA,
        pltpu.SemaphoreType.DMA,
        pltpu.VMEM((48,), jnp.int32),
        pltpu.VMEM((16,), jnp.float32),
        pltpu.VMEM((NPN,), jnp.int32),
        pltpu.VMEM((NPN,), jnp.int32),
        pltpu.VMEM((NPN,), jnp.int32),
        pltpu.VMEM((NPN,), jnp.int32),
        pltpu.VMEM((NPN,), jnp.int32),
        pltpu.VMEM((NPN,), jnp.int32),
        pltpu.VMEM((NPN,), jnp.int32),
        pltpu.VMEM((NPN,), jnp.int32),
        pltpu.VMEM((NPN,), jnp.float32),
        pltpu.VMEM((NPN,), jnp.float32),
        pltpu.VMEM((NPN,), jnp.float32),
        pltpu.VMEM((NPN,), jnp.float32),
        pltpu.VMEM((NPN,), jnp.float32),
        pltpu.VMEM((NPN,), jnp.float32),
        pltpu.VMEM((NPN,), jnp.float32),
        pltpu.VMEM((NPN,), jnp.float32),
        pltpu.VMEM((NPN,), jnp.float32),
        pltpu.VMEM((NPN,), jnp.float32),
        pltpu.VMEM((NPN,), jnp.float32),
        pltpu.VMEM((NPN,), jnp.float32),
    ])
def _sc_tables(x_hbm, y_hbm, id_hbm, st_hbm, par_hbm, out_hbm,
               xb0, xb1, yb0, yb1, ib0, ib1, sem0, sem1, svec, pvec,
               qa0, qa1, qa2, qa3, qb0, qb1, qb2, qb3,
               r0, r1, r2, r3,
               sa0, sa1, sa2, sa3, sb0, sb1, sb2, sb3):
    c = lax.axis_index("c")
    s = lax.axis_index("s")
    wid = s * 2 + c

    pltpu.sync_copy(st_hbm, svec)
    pltpu.sync_copy(par_hbm, pvec)
    invg = pvec[...]

    iota = lax.iota(jnp.int32, 16)
    ip = jnp.maximum(iota - 1, 0)
    inx = jnp.minimum(iota + 1, 15)

    v0 = svec[pl.ds(0, 16)]
    v1 = svec[pl.ds(16, 16)]
    v2 = svec[pl.ds(32, 16)]

    def pick(w):
        z = (jnp.where(iota == w, v0, 0) + jnp.where(iota == w - 16, v1, 0)
             + jnp.where(iota == w - 32, v2, 0))
        return jnp.sum(z.astype(jnp.float32)).astype(jnp.int32)

    p_lo = pick(wid)
    p_hi = pick(wid + 1)
    net_lo = wid * NPN
    p0 = p_lo & jnp.int32(-8)
    nch = (p_hi - p0 + (CHUNK - 1)) // CHUNK

    zi = jnp.zeros((16,), jnp.int32)
    zf = jnp.zeros((16,), jnp.float32)

    def zbody(i, carry):
        sl = pl.ds(i * 16, 16)
        for t in (qa0, qa1, qa2, qa3, qb0, qb1, qb2, qb3):
            t[sl] = zi
        for t in (sa0, sa1, sa2, sa3, sb0, sb1, sb2, sb3):
            t[sl] = zf
        return carry
    lax.fori_loop(0, NPN // 16, zbody, 0)

    slots = ((xb0, yb0, ib0, sem0), (xb1, yb1, ib1, sem1))

    def off_of(k):
        return pl.multiple_of(p0 + k * CHUNK, 8)

    def start(slot, off):
        xb, yb, ib, sem = slots[slot]
        pltpu.async_copy(x_hbm.at[pl.ds(off, CHUNK)], xb, sem)
        pltpu.async_copy(y_hbm.at[pl.ds(off, CHUNK)], yb, sem)
        pltpu.async_copy(id_hbm.at[pl.ds(off, CHUNK)], ib, sem)

    def wait(slot):
        xb, yb, ib, sem = slots[slot]
        pltpu.make_async_copy(x_hbm.at[pl.ds(0, CHUNK)], xb, sem).wait()
        pltpu.make_async_copy(y_hbm.at[pl.ds(0, CHUNK)], yb, sem).wait()
        pltpu.make_async_copy(id_hbm.at[pl.ds(0, CHUNK)], ib, sem).wait()

    def run_pass(process_vreg):
        start(0, off_of(0))

        def pair_body(p, carry):
            k0 = 2 * p
            start(1, off_of(k0 + 1))
            wait(0)

            def vb0(j, cc):
                process_vreg(slots[0][0], slots[0][1], slots[0][2],
                             2 * j, 0)
                process_vreg(slots[0][0], slots[0][1], slots[0][2],
                             2 * j + 1, 1)
                return cc
            lax.fori_loop(0, CHUNK // 32, vb0, 0)
            start(0, off_of(k0 + 2))
            wait(1)

            def vb1(j, cc):
                process_vreg(slots[1][0], slots[1][1], slots[1][2],
                             2 * j, 0)
                process_vreg(slots[1][0], slots[1][1], slots[1][2],
                             2 * j + 1, 1)
                return cc
            lax.fori_loop(0, CHUNK // 32, vb1, 0)
            return carry
        lax.fori_loop(0, (nch + 1) // 2, pair_body, 0)
        wait(0)

    qsets = ((qa0, qa1, qa2, qa3), (qb0, qb1, qb2, qb3))
    ssets = ((sa0, sa1, sa2, sa3), (sb0, sb1, sb2, sb3))

    def vreg_a(xbuf, ybuf, idbuf, j, tset):
        sl = pl.ds(j * 16, 16)
        ids = idbuf[sl]
        xv = xbuf[sl]
        yv = ybuf[sl]
        relu = plsc.bitcast(ids - net_lo, jnp.uint32)
        inb = relu < NPN
        idx = plsc.bitcast(jnp.minimum(relu, NPN - 1), jnp.int32)
        tail = (iota == 15) | (ids != _vshift(ids, inx))
        wmask = tail & inb
        hi = jnp.left_shift(relu, 19)
        bx = jnp.right_shift(plsc.bitcast(xv, jnp.uint32), 12)
        by = jnp.right_shift(plsc.bitcast(yv, jnp.uint32), 12)
        qs = qsets[tset]
        for tab, b in ((qs[0], bx), (qs[1], MASK19 - bx),
                       (qs[2], by), (qs[3], MASK19 - by)):
            comb = jnp.where(inb, hi | b, jnp.uint32(0))
            cm = plsc.bitcast(plsc.cummax(comb), jnp.int32)
            old = plsc.load_gather(tab, [idx])
            plsc.store_scatter(tab, [idx], jnp.maximum(old, cm), mask=wmask)

    run_pass(vreg_a)

    def refbody(i, carry):
        sl = pl.ds(i * 16, 16)
        for qta, qtb, rt, minus in ((qa0, qb0, r0, False),
                                    (qa1, qb1, r1, True),
                                    (qa2, qb2, r2, False),
                                    (qa3, qb3, r3, True)):
            q = jnp.maximum(qta[sl], qtb[sl]) & MASK19
            b = (MASK19 - q) if minus else q
            r = plsc.bitcast(jnp.left_shift(b, 12), jnp.float32)
            rt[sl] = r * invg
        return carry
    lax.fori_loop(0, NPN // 16, refbody, 0)

    def vreg_c(xbuf, ybuf, idbuf, j, tset):
        sl = pl.ds(j * 16, 16)
        ids = idbuf[sl]
        xv = xbuf[sl]
        yv = ybuf[sl]
        relu = plsc.bitcast(ids - net_lo, jnp.uint32)
        inb = relu < NPN
        idx = plsc.bitcast(jnp.minimum(relu, NPN - 1), jnp.int32)
        tail = (iota == 15) | (ids != _vshift(ids, inx))
        head = (iota == 0) | (ids != _vshift(ids, ip))
        tmask = tail & inb
        hmask = head & inb
        xg = xv * invg
        yg = yv * invg
        ss = ssets[tset]
        for st, rt, neg, vg in ((ss[0], r0, False, xg), (ss[1], r1, True, xg),
                                (ss[2], r2, False, yg), (ss[3], r3, True, yg)):
            rg = plsc.load_gather(rt, [idx])
            arg = (rg - vg) if neg else (vg - rg)
            e = jnp.where(inb, jnp.exp(arg), 0.0)
            cs = jnp.cumsum(e)
            pfx = jnp.where(iota == 0, 0.0, _vshift(cs, ip))
            plsc.addupdate_scatter(st, [idx], cs, mask=tmask)
            plsc.addupdate_scatter(st, [idx], -pfx, mask=hmask)

    run_pass(vreg_c)

    def mbody(i, carry):
        sl = pl.ds(i * 16, 16)
        for ta, tb in zip((sa0, sa1, sa2, sa3), (sb0, sb1, sb2, sb3)):
            ta[sl] = ta[sl] + tb[sl]
        return carry
    lax.fori_loop(0, NPN // 16, mbody, 0)

    base = wid * NPN
    for row, t in enumerate((sa0, sa1, sa2, sa3, r0, r1, r2, r3)):
        pltpu.sync_copy(
            t, out_hbm.at[pl.ds(pl.multiple_of(row * NPAD + base, 8), NPN)])


def _tc_final(tab_ref, mask_ref, g_ref, out_ref):
    g = g_ref[0, 0]
    sp_x = tab_ref[0]
    sm_x = tab_ref[1]
    sp_y = tab_ref[2]
    sm_y = tab_ref[3]
    rp_x = tab_ref[4]
    rm_x = tab_ref[5]
    rp_y = tab_ref[6]
    rm_y = tab_ref[7]
    m = mask_ref[...]
    valid = (sp_x > 0.0) & (m > 0.0)
    one = jnp.float32(1.0)

    def lg(sv):
        return jnp.log(jnp.where(valid, sv, one))

    tot = jnp.where(
        valid,
        (rp_x - rm_x) + (rp_y - rm_y)
        + lg(sp_x) + lg(sm_x) + lg(sp_y) + lg(sm_y),
        0.0)
    out_ref[0, 0] = g * jnp.sum(tot)


def kernel(pos, pin2net_map, net_mask, gamma):
    x = pos[:NPIN]
    y = pos[NPIN:]
    bounds = jnp.arange(0, (NSUB + 1) * NPN, NPN, dtype=jnp.int32)
    starts = jnp.searchsorted(pin2net_map, bounds).astype(jnp.int32)
    st = jnp.zeros((48,), jnp.int32).at[:33].set(starts)
    xp = jnp.concatenate([x, jnp.zeros((PAD,), jnp.float32)])
    yp = jnp.concatenate([y, jnp.zeros((PAD,), jnp.float32)])
    idp = jnp.concatenate([pin2net_map,
                           jnp.full((PAD,), SENT, jnp.int32)])
    g = jnp.asarray(gamma, jnp.float32)
    par = jnp.full((16,), 1.0, jnp.float32) / g

    tabs = _sc_tables(xp, yp, idp, st, par)

    mask = jnp.concatenate([net_mask.astype(jnp.float32),
                            jnp.zeros((NPAD - NNET,), jnp.float32)])
    res = pl.pallas_call(
        _tc_final,
        out_shape=jax.ShapeDtypeStruct((1, 1), jnp.float32),
        in_specs=[
            pl.BlockSpec(memory_space=pltpu.VMEM),
            pl.BlockSpec(memory_space=pltpu.VMEM),
            pl.BlockSpec(memory_space=pltpu.SMEM),
        ],
        out_specs=pl.BlockSpec(memory_space=pltpu.SMEM),
    )(tabs.reshape(8, 800, 128), mask.reshape(800, 128), g.reshape(1, 1))
    return res[0, 0]


# speedup vs baseline: 1.0672x; 1.0672x over previous
import functools

import jax
import jax.numpy as jnp
from jax import lax
from jax.experimental import pallas as pl
from jax.experimental.pallas import tpu as pltpu
from jax.experimental.pallas import tpu_sc as plsc

NPIN = 3200000
NNET = 100000
NSUB = 32
NPN = 3200
NPAD = NSUB * NPN
CHUNK = 4096
PAD = 3 * CHUNK
MASK19 = (1 << 19) - 1
SENT = 1 << 20


def _vshift(v, idx):
    return lax.gather(
        v, idx[:, None],
        lax.GatherDimensionNumbers(offset_dims=(), collapsed_slice_dims=(0,),
                                   start_index_map=(0,)),
        (1,), mode=lax.GatherScatterMode.PROMISE_IN_BOUNDS)


_mesh = plsc.VectorSubcoreMesh(core_axis_name="c", subcore_axis_name="s")


@functools.partial(
    pl.kernel,
    out_type=jax.ShapeDtypeStruct((8 * NPAD,), jnp.float32),
    mesh=_mesh,
    compiler_params=pltpu.CompilerParams(needs_layout_passes=False),
    scratch_types=[
        pltpu.VMEM((CHUNK,), jnp.float32),
        pltpu.VMEM((CHUNK,), jnp.float32),
        pltpu.VMEM((CHUNK,), jnp.float32),
        pltpu.VMEM((CHUNK,), jnp.float32),
        pltpu.VMEM((CHUNK,), jnp.int32),
        pltpu.VMEM((CHUNK,), jnp.int32),
        pltpu.SemaphoreType.DMA,
        pltpu.SemaphoreType.DMA,
        pltpu.VMEM((48,), jnp.int32),
        pltpu.VMEM((16,), jnp.float32),
        pltpu.VMEM((NPN,), jnp.int32),
        pltpu.VMEM((NPN,), jnp.int32),
        pltpu.VMEM((NPN,), jnp.int32),
        pltpu.VMEM((NPN,), jnp.int32),
        pltpu.VMEM((NPN,), jnp.int32),
        pltpu.VMEM((NPN,), jnp.int32),
        pltpu.VMEM((NPN,), jnp.int32),
        pltpu.VMEM((NPN,), jnp.int32),
        pltpu.VMEM((NPN,), jnp.float32),
        pltpu.VMEM((NPN,), jnp.float32),
        pltpu.VMEM((NPN,), jnp.float32),
        pltpu.VMEM((NPN,), jnp.float32),
        pltpu.VMEM((NPN,), jnp.float32),
        pltpu.VMEM((NPN,), jnp.float32),
        pltpu.VMEM((NPN,), jnp.float32),
        pltpu.VMEM((NPN,), jnp.float32),
        pltpu.VMEM((NPN,), jnp.float32),
        pltpu.VMEM((NPN,), jnp.float32),
        pltpu.VMEM((NPN,), jnp.float32),
        pltpu.VMEM((NPN,), jnp.float32),
    ])
def _sc_tables(x_hbm, y_hbm, id_hbm, st_hbm, par_hbm, out_hbm,
               xb0, xb1, yb0, yb1, ib0, ib1, sem0, sem1, svec, pvec,
               qa0, qa1, qa2, qa3, qb0, qb1, qb2, qb3,
               r0, r1, r2, r3,
               sa0, sa1, sa2, sa3, sb0, sb1, sb2, sb3):
    c = lax.axis_index("c")
    s = lax.axis_index("s")
    wid = s * 2 + c

    pltpu.sync_copy(st_hbm, svec)
    pltpu.sync_copy(par_hbm, pvec)
    invg = pvec[...]

    iota = lax.iota(jnp.int32, 16)
    ip = jnp.maximum(iota - 1, 0)
    inx = jnp.minimum(iota + 1, 15)

    v0 = svec[pl.ds(0, 16)]
    v1 = svec[pl.ds(16, 16)]
    v2 = svec[pl.ds(32, 16)]

    def pick(w):
        z = (jnp.where(iota == w, v0, 0) + jnp.where(iota == w - 16, v1, 0)
             + jnp.where(iota == w - 32, v2, 0))
        return jnp.sum(z.astype(jnp.float32)).astype(jnp.int32)

    p_lo = pick(wid)
    p_hi = pick(wid + 1)
    net_lo = wid * NPN
    p0 = p_lo & jnp.int32(-8)
    nch = (p_hi - p0 + (CHUNK - 1)) // CHUNK

    zi = jnp.zeros((16,), jnp.int32)
    zf = jnp.zeros((16,), jnp.float32)

    def zbody(i, carry):
        sl = pl.ds(i * 16, 16)
        for t in (qa0, qa1, qa2, qa3, qb0, qb1, qb2, qb3):
            t[sl] = zi
        for t in (sa0, sa1, sa2, sa3, sb0, sb1, sb2, sb3):
            t[sl] = zf
        return carry
    lax.fori_loop(0, NPN // 16, zbody, 0)

    slots = ((xb0, yb0, ib0, sem0), (xb1, yb1, ib1, sem1))

    def off_of(k):
        return pl.multiple_of(p0 + k * CHUNK, 8)

    def start(slot, off):
        xb, yb, ib, sem = slots[slot]
        pltpu.async_copy(x_hbm.at[pl.ds(off, CHUNK)], xb, sem)
        pltpu.async_copy(y_hbm.at[pl.ds(off, CHUNK)], yb, sem)
        pltpu.async_copy(id_hbm.at[pl.ds(off, CHUNK)], ib, sem)

    def wait(slot):
        xb, yb, ib, sem = slots[slot]
        pltpu.make_async_copy(x_hbm.at[pl.ds(0, CHUNK)], xb, sem).wait()
        pltpu.make_async_copy(y_hbm.at[pl.ds(0, CHUNK)], yb, sem).wait()
        pltpu.make_async_copy(id_hbm.at[pl.ds(0, CHUNK)], ib, sem).wait()

    def run_pass(process_vreg):
        start(0, off_of(0))

        def pair_body(p, carry):
            k0 = 2 * p
            start(1, off_of(k0 + 1))
            wait(0)

            def vb0(j, cc):
                process_vreg(slots[0][0], slots[0][1], slots[0][2],
                             2 * j, 0)
                process_vreg(slots[0][0], slots[0][1], slots[0][2],
                             2 * j + 1, 1)
                return cc
            lax.fori_loop(0, CHUNK // 32, vb0, 0)
            start(0, off_of(k0 + 2))
            wait(1)

            def vb1(j, cc):
                process_vreg(slots[1][0], slots[1][1], slots[1][2],
                             2 * j, 0)
                process_vreg(slots[1][0], slots[1][1], slots[1][2],
                             2 * j + 1, 1)
                return cc
            lax.fori_loop(0, CHUNK // 32, vb1, 0)
            return carry
        lax.fori_loop(0, (nch + 1) // 2, pair_body, 0)
        wait(0)

    qsets = ((qa0, qa1, qa2, qa3), (qb0, qb1, qb2, qb3))
    ssets = ((sa0, sa1, sa2, sa3), (sb0, sb1, sb2, sb3))

    def vreg_a(xbuf, ybuf, idbuf, j, tset):
        sl = pl.ds(j * 16, 16)
        ids = idbuf[sl]
        xv = xbuf[sl]
        yv = ybuf[sl]
        relu = plsc.bitcast(ids - net_lo, jnp.uint32)
        inb = relu < NPN
        idx = plsc.bitcast(jnp.minimum(relu, NPN - 1), jnp.int32)
        tail = (iota == 15) | (ids != _vshift(ids, inx))
        wmask = tail & inb
        hi = jnp.left_shift(relu, 19)
        bx = jnp.right_shift(plsc.bitcast(xv, jnp.uint32), 12)
        by = jnp.right_shift(plsc.bitcast(yv, jnp.uint32), 12)
        qs = qsets[tset]
        for tab, b in ((qs[0], bx), (qs[1], MASK19 - bx),
                       (qs[2], by), (qs[3], MASK19 - by)):
            comb = jnp.where(inb, hi | b, jnp.uint32(0))
            cm = plsc.bitcast(plsc.cummax(comb), jnp.int32)
            old = plsc.load_gather(tab, [idx])
            plsc.store_scatter(tab, [idx], jnp.maximum(old, cm), mask=wmask)

    run_pass(vreg_a)

    def refbody(i, carry):
        sl = pl.ds(i * 16, 16)
        for qta, qtb, rt, minus in ((qa0, qb0, r0, False),
                                    (qa1, qb1, r1, True),
                                    (qa2, qb2, r2, False),
                                    (qa3, qb3, r3, True)):
            q = jnp.maximum(qta[sl], qtb[sl]) & MASK19
            b = (MASK19 - q) if minus else q
            r = plsc.bitcast(jnp.left_shift(b, 12), jnp.float32)
            rt[sl] = r * invg
        return carry
    lax.fori_loop(0, NPN // 16, refbody, 0)

    def vreg_c(xbuf, ybuf, idbuf, j, tset):
        sl = pl.ds(j * 16, 16)
        ids = idbuf[sl]
        xv = xbuf[sl]
        yv = ybuf[sl]
        relu = plsc.bitcast(ids - net_lo, jnp.uint32)
        inb = relu < NPN
        idx = plsc.bitcast(jnp.minimum(relu, NPN - 1), jnp.int32)
        tail = (iota == 15) | (ids != _vshift(ids, inx))
        head = (iota == 0) | (ids != _vshift(ids, ip))
        tmask = tail & inb
        hmask = head & inb
        xg = xv * invg
        yg = yv * invg
        ss = ssets[tset]
        for st, rt, neg, vg in ((ss[0], r0, False, xg), (ss[1], r1, True, xg),
                                (ss[2], r2, False, yg), (ss[3], r3, True, yg)):
            rg = plsc.load_gather(rt, [idx])
            arg = (rg - vg) if neg else (vg - rg)
            e = jnp.where(inb, jnp.exp(arg), 0.0)
            cs = jnp.cumsum(e)
            pfx = jnp.where(iota == 0, 0.0, _vshift(cs, ip))
            plsc.addupdate_scatter(st, [idx], cs, mask=tmask)
            plsc.addupdate_scatter(st, [idx], -pfx, mask=hmask)

    run_pass(vreg_c)

    def mbody(i, carry):
        sl = pl.ds(i * 16, 16)
        for ta, tb in zip((sa0, sa1, sa2, sa3), (sb0, sb1, sb2, sb3)):
            ta[sl] = ta[sl] + tb[sl]
        return carry
    lax.fori_loop(0, NPN // 16, mbody, 0)

    base = wid * NPN
    for row, t in enumerate((sa0, sa1, sa2, sa3, r0, r1, r2, r3)):
        pltpu.sync_copy(
            t, out_hbm.at[pl.ds(pl.multiple_of(row * NPAD + base, 8), NPN)])


def _tc_final(tab_ref, mask_ref, g_ref, out_ref):
    g = g_ref[0, 0]
    sp_x = tab_ref[0]
    sm_x = tab_ref[1]
    sp_y = tab_ref[2]
    sm_y = tab_ref[3]
    rp_x = tab_ref[4]
    rm_x = tab_ref[5]
    rp_y = tab_ref[6]
    rm_y = tab_ref[7]
    m = mask_ref[...]
    valid = (sp_x > 0.0) & (m > 0.0)
    one = jnp.float32(1.0)

    def lg(sv):
        return jnp.log(jnp.where(valid, sv, one))

    tot = jnp.where(
        valid,
        (rp_x - rm_x) + (rp_y - rm_y)
        + lg(sp_x) + lg(sm_x) + lg(sp_y) + lg(sm_y),
        0.0)
    out_ref[0, 0] = g * jnp.sum(tot)


def kernel(pos, pin2net_map, net_mask, gamma):
    x = pos[:NPIN]
    y = pos[NPIN:]
    bounds = jnp.arange(0, (NSUB + 1) * NPN, NPN, dtype=jnp.int32)
    starts = jnp.searchsorted(pin2net_map, bounds).astype(jnp.int32)
    st = jnp.zeros((48,), jnp.int32).at[:33].set(starts)
    xp = jnp.concatenate([x, jnp.zeros((PAD,), jnp.float32)])
    yp = jnp.concatenate([y, jnp.zeros((PAD,), jnp.float32)])
    idp = jnp.concatenate([pin2net_map,
                           jnp.full((PAD,), SENT, jnp.int32)])
    g = jnp.asarray(gamma, jnp.float32)
    par = jnp.full((16,), 1.0, jnp.float32) / g

    tabs = _sc_tables(xp, yp, idp, st, par)

    mask = jnp.concatenate([net_mask.astype(jnp.float32),
                            jnp.zeros((NPAD - NNET,), jnp.float32)])
    res = pl.pallas_call(
        _tc_final,
        out_shape=jax.ShapeDtypeStruct((1, 1), jnp.float32),
        in_specs=[
            pl.BlockSpec(memory_space=pltpu.VMEM),
            pl.BlockSpec(memory_space=pltpu.VMEM),
            pl.BlockSpec(memory_space=pltpu.SMEM),
        ],
        out_specs=pl.BlockSpec(memory_space=pltpu.SMEM),
    )(tabs.reshape(8, 800, 128), mask.reshape(800, 128), g.reshape(1, 1))
    return res[0, 0]


# speedup vs baseline: 1.0792x; 1.0112x over previous
import functools

import jax
import jax.numpy as jnp
from jax import lax
from jax.experimental import pallas as pl
from jax.experimental.pallas import tpu as pltpu
from jax.experimental.pallas import tpu_sc as plsc

NPIN = 3200000
NNET = 100000
NSUB = 32
NPN = 3200
NPAD = NSUB * NPN
CHUNK = 4096
PAD = 3 * CHUNK
MASK19 = (1 << 19) - 1
SENT = 1 << 20


def _vshift(v, idx):
    return lax.gather(
        v, idx[:, None],
        lax.GatherDimensionNumbers(offset_dims=(), collapsed_slice_dims=(0,),
                                   start_index_map=(0,)),
        (1,), mode=lax.GatherScatterMode.PROMISE_IN_BOUNDS)


_mesh = plsc.VectorSubcoreMesh(core_axis_name="c", subcore_axis_name="s")


@functools.partial(
    pl.kernel,
    out_type=jax.ShapeDtypeStruct((8 * NPAD,), jnp.float32),
    mesh=_mesh,
    compiler_params=pltpu.CompilerParams(needs_layout_passes=False),
    scratch_types=[
        pltpu.VMEM((CHUNK,), jnp.float32),
        pltpu.VMEM((CHUNK,), jnp.float32),
        pltpu.VMEM((CHUNK,), jnp.float32),
        pltpu.VMEM((CHUNK,), jnp.float32),
        pltpu.VMEM((CHUNK,), jnp.int32),
        pltpu.VMEM((CHUNK,), jnp.int32),
        pltpu.SemaphoreType.DMA,
        pltpu.SemaphoreType.DMA,
        pltpu.VMEM((48,), jnp.int32),
        pltpu.VMEM((16,), jnp.float32),
        pltpu.VMEM((NPN,), jnp.int32),
        pltpu.VMEM((NPN,), jnp.int32),
        pltpu.VMEM((NPN,), jnp.int32),
        pltpu.VMEM((NPN,), jnp.int32),
        pltpu.VMEM((NPN,), jnp.int32),
        pltpu.VMEM((NPN,), jnp.int32),
        pltpu.VMEM((NPN,), jnp.int32),
        pltpu.VMEM((NPN,), jnp.int32),
        pltpu.VMEM((NPN,), jnp.float32),
        pltpu.VMEM((NPN,), jnp.float32),
        pltpu.VMEM((NPN,), jnp.float32),
        pltpu.VMEM((NPN,), jnp.float32),
        pltpu.VMEM((NPN,), jnp.float32),
        pltpu.VMEM((NPN,), jnp.float32),
        pltpu.VMEM((NPN,), jnp.float32),
        pltpu.VMEM((NPN,), jnp.float32),
        pltpu.VMEM((NPN,), jnp.float32),
        pltpu.VMEM((NPN,), jnp.float32),
        pltpu.VMEM((NPN,), jnp.float32),
        pltpu.VMEM((NPN,), jnp.float32),
    ])
def _sc_tables(x_hbm, y_hbm, id_hbm, st_hbm, par_hbm, out_hbm,
               xb0, xb1, yb0, yb1, ib0, ib1, sem0, sem1, svec, pvec,
               qa0, qa1, qa2, qa3, qb0, qb1, qb2, qb3,
               r0, r1, r2, r3,
               sa0, sa1, sa2, sa3, sb0, sb1, sb2, sb3):
    c = lax.axis_index("c")
    s = lax.axis_index("s")
    wid = s * 2 + c

    pltpu.sync_copy(st_hbm, svec)
    pltpu.sync_copy(par_hbm, pvec)
    invg = pvec[...]

    iota = lax.iota(jnp.int32, 16)
    ip = jnp.maximum(iota - 1, 0)
    inx = jnp.minimum(iota + 1, 15)

    v0 = svec[pl.ds(0, 16)]
    v1 = svec[pl.ds(16, 16)]
    v2 = svec[pl.ds(32, 16)]

    def pick(w):
        z = (jnp.where(iota == w, v0, 0) + jnp.where(iota == w - 16, v1, 0)
             + jnp.where(iota == w - 32, v2, 0))
        return jnp.sum(z.astype(jnp.float32)).astype(jnp.int32)

    p_lo = pick(wid)
    p_hi = pick(wid + 1)
    net_lo = wid * NPN
    p0 = p_lo & jnp.int32(-8)
    nch = (p_hi - p0 + (CHUNK - 1)) // CHUNK

    zi = jnp.zeros((16,), jnp.int32)
    zf = jnp.zeros((16,), jnp.float32)

    def zbody(i, carry):
        sl = pl.ds(i * 16, 16)
        for t in (qa0, qa1, qa2, qa3, qb0, qb1, qb2, qb3):
            t[sl] = zi
        for t in (sa0, sa1, sa2, sa3, sb0, sb1, sb2, sb3):
            t[sl] = zf
        return carry
    lax.fori_loop(0, NPN // 16, zbody, 0)

    slots = ((xb0, yb0, ib0, sem0), (xb1, yb1, ib1, sem1))

    def off_of(k):
        return pl.multiple_of(p0 + k * CHUNK, 8)

    def start(slot, off):
        xb, yb, ib, sem = slots[slot]
        pltpu.async_copy(x_hbm.at[pl.ds(off, CHUNK)], xb, sem)
        pltpu.async_copy(y_hbm.at[pl.ds(off, CHUNK)], yb, sem)
        pltpu.async_copy(id_hbm.at[pl.ds(off, CHUNK)], ib, sem)

    def wait(slot):
        xb, yb, ib, sem = slots[slot]
        pltpu.make_async_copy(x_hbm.at[pl.ds(0, CHUNK)], xb, sem).wait()
        pltpu.make_async_copy(y_hbm.at[pl.ds(0, CHUNK)], yb, sem).wait()
        pltpu.make_async_copy(id_hbm.at[pl.ds(0, CHUNK)], ib, sem).wait()

    def run_pass(process_vreg):
        start(0, off_of(0))

        def pair_body(p, carry):
            k0 = 2 * p
            start(1, off_of(k0 + 1))
            wait(0)

            def vb0(j, cc):
                process_vreg(slots[0][0], slots[0][1], slots[0][2],
                             2 * j, 0)
                process_vreg(slots[0][0], slots[0][1], slots[0][2],
                             2 * j + 1, 1)
                return cc
            lax.fori_loop(0, CHUNK // 32, vb0, 0)
            start(0, off_of(k0 + 2))
            wait(1)

            def vb1(j, cc):
                process_vreg(slots[1][0], slots[1][1], slots[1][2],
                             2 * j, 0)
                process_vreg(slots[1][0], slots[1][1], slots[1][2],
                             2 * j + 1, 1)
                return cc
            lax.fori_loop(0, CHUNK // 32, vb1, 0)
            return carry
        lax.fori_loop(0, (nch + 1) // 2, pair_body, 0)
        wait(0)

    qsets = ((qa0, qa1, qa2, qa3), (qb0, qb1, qb2, qb3))
    ssets = ((sa0, sa1, sa2, sa3), (sb0, sb1, sb2, sb3))

    def vreg_a(xbuf, ybuf, idbuf, j, tset):
        sl = pl.ds(j * 16, 16)
        ids = idbuf[sl]
        xv = xbuf[sl]
        yv = ybuf[sl]
        relu = plsc.bitcast(ids - net_lo, jnp.uint32)
        inb = relu < NPN
        idx = plsc.bitcast(jnp.minimum(relu, NPN - 1), jnp.int32)
        tail = (iota == 15) | (ids != _vshift(ids, inx))
        wmask = tail & inb
        hi = jnp.left_shift(relu, 19)
        bx = jnp.right_shift(plsc.bitcast(xv, jnp.uint32), 12)
        by = jnp.right_shift(plsc.bitcast(yv, jnp.uint32), 12)
        qs = qsets[tset]
        for tab, b in ((qs[0], bx), (qs[1], MASK19 - bx),
                       (qs[2], by), (qs[3], MASK19 - by)):
            comb = jnp.where(inb, hi | b, jnp.uint32(0))
            cm = plsc.bitcast(plsc.cummax(comb), jnp.int32)
            old = plsc.load_gather(tab, [idx])
            plsc.store_scatter(tab, [idx], jnp.maximum(old, cm), mask=wmask)

    run_pass(vreg_a)

    def refbody(i, carry):
        sl = pl.ds(i * 16, 16)
        for qta, qtb, rt, minus in ((qa0, qb0, r0, False),
                                    (qa1, qb1, r1, True),
                                    (qa2, qb2, r2, False),
                                    (qa3, qb3, r3, True)):
            q = jnp.maximum(qta[sl], qtb[sl]) & MASK19
            b = (MASK19 - q) if minus else q
            r = plsc.bitcast(jnp.left_shift(b, 12), jnp.float32)
            rt[sl] = r * invg
        return carry
    lax.fori_loop(0, NPN // 16, refbody, 0)

    def vreg_c(xbuf, ybuf, idbuf, j, tset):
        sl = pl.ds(j * 16, 16)
        ids = idbuf[sl]
        xv = xbuf[sl]
        yv = ybuf[sl]
        relu = plsc.bitcast(ids - net_lo, jnp.uint32)
        inb = relu < NPN
        idx = plsc.bitcast(jnp.minimum(relu, NPN - 1), jnp.int32)
        tail = (iota == 15) | (ids != _vshift(ids, inx))
        head = ids != _vshift(ids, ip)
        tmask = tail & inb
        h = plsc.cummax(jnp.where(head, iota, 0))
        hm1 = jnp.maximum(h - 1, 0)
        hz = h == 0
        xg = xv * invg
        yg = yv * invg
        ss = ssets[tset]
        for st, rt, neg, vg in ((ss[0], r0, False, xg), (ss[1], r1, True, xg),
                                (ss[2], r2, False, yg), (ss[3], r3, True, yg)):
            rg = plsc.load_gather(rt, [idx])
            arg = (rg - vg) if neg else (vg - rg)
            e = jnp.where(inb, jnp.exp(arg), 0.0)
            cs = jnp.cumsum(e)
            pfx = jnp.where(hz, 0.0, _vshift(cs, hm1))
            plsc.addupdate_scatter(st, [idx], cs - pfx, mask=tmask)

    run_pass(vreg_c)

    def mbody(i, carry):
        sl = pl.ds(i * 16, 16)
        for ta, tb in zip((sa0, sa1, sa2, sa3), (sb0, sb1, sb2, sb3)):
            ta[sl] = ta[sl] + tb[sl]
        return carry
    lax.fori_loop(0, NPN // 16, mbody, 0)

    base = wid * NPN
    for row, t in enumerate((sa0, sa1, sa2, sa3, r0, r1, r2, r3)):
        pltpu.sync_copy(
            t, out_hbm.at[pl.ds(pl.multiple_of(row * NPAD + base, 8), NPN)])


def _tc_final(tab_ref, mask_ref, g_ref, out_ref):
    g = g_ref[0, 0]
    sp_x = tab_ref[0]
    sm_x = tab_ref[1]
    sp_y = tab_ref[2]
    sm_y = tab_ref[3]
    rp_x = tab_ref[4]
    rm_x = tab_ref[5]
    rp_y = tab_ref[6]
    rm_y = tab_ref[7]
    m = mask_ref[...]
    valid = (sp_x > 0.0) & (m > 0.0)
    one = jnp.float32(1.0)

    def lg(sv):
        return jnp.log(jnp.where(valid, sv, one))

    tot = jnp.where(
        valid,
        (rp_x - rm_x) + (rp_y - rm_y)
        + lg(sp_x) + lg(sm_x) + lg(sp_y) + lg(sm_y),
        0.0)
    out_ref[0, 0] = g * jnp.sum(tot)


def kernel(pos, pin2net_map, net_mask, gamma):
    x = pos[:NPIN]
    y = pos[NPIN:]
    bounds = jnp.arange(0, (NSUB + 1) * NPN, NPN, dtype=jnp.int32)
    starts = jnp.searchsorted(pin2net_map, bounds).astype(jnp.int32)
    st = jnp.zeros((48,), jnp.int32).at[:33].set(starts)
    xp = jnp.concatenate([x, jnp.zeros((PAD,), jnp.float32)])
    yp = jnp.concatenate([y, jnp.zeros((PAD,), jnp.float32)])
    idp = jnp.concatenate([pin2net_map,
                           jnp.full((PAD,), SENT, jnp.int32)])
    g = jnp.asarray(gamma, jnp.float32)
    par = jnp.full((16,), 1.0, jnp.float32) / g

    tabs = _sc_tables(xp, yp, idp, st, par)

    mask = jnp.concatenate([net_mask.astype(jnp.float32),
                            jnp.zeros((NPAD - NNET,), jnp.float32)])
    res = pl.pallas_call(
        _tc_final,
        out_shape=jax.ShapeDtypeStruct((1, 1), jnp.float32),
        in_specs=[
            pl.BlockSpec(memory_space=pltpu.VMEM),
            pl.BlockSpec(memory_space=pltpu.VMEM),
            pl.BlockSpec(memory_space=pltpu.SMEM),
        ],
        out_specs=pl.BlockSpec(memory_space=pltpu.SMEM),
    )(tabs.reshape(8, 800, 128), mask.reshape(800, 128), g.reshape(1, 1))
    return res[0, 0]


# speedup vs baseline: 1.1591x; 1.0740x over previous
import functools

import jax
import jax.numpy as jnp
from jax import lax
from jax.experimental import pallas as pl
from jax.experimental.pallas import tpu as pltpu
from jax.experimental.pallas import tpu_sc as plsc

NPIN = 3200000
NNET = 100000
NSUB = 32
NPN = 3200
NPAD = NSUB * NPN
CHUNK = 4096
PAD = 3 * CHUNK
MASK19 = (1 << 19) - 1
SENT = 1 << 20
FLO = -3.4e38
FHI = 3.4e38


def _vshift(v, idx):
    return lax.gather(
        v, idx[:, None],
        lax.GatherDimensionNumbers(offset_dims=(), collapsed_slice_dims=(0,),
                                   start_index_map=(0,)),
        (1,), mode=lax.GatherScatterMode.PROMISE_IN_BOUNDS)


_mesh = plsc.VectorSubcoreMesh(core_axis_name="c", subcore_axis_name="s")


@functools.partial(
    pl.kernel,
    out_type=jax.ShapeDtypeStruct((16 * NPAD,), jnp.float32),
    mesh=_mesh,
    compiler_params=pltpu.CompilerParams(needs_layout_passes=False),
    scratch_types=[
        pltpu.VMEM((CHUNK,), jnp.float32),
        pltpu.VMEM((CHUNK,), jnp.float32),
        pltpu.VMEM((CHUNK,), jnp.float32),
        pltpu.VMEM((CHUNK,), jnp.float32),
        pltpu.VMEM((CHUNK,), jnp.int32),
        pltpu.VMEM((CHUNK,), jnp.int32),
        pltpu.SemaphoreType.DMA,
        pltpu.SemaphoreType.DMA,
        pltpu.VMEM((48,), jnp.int32),
        pltpu.VMEM((16,), jnp.float32),
        pltpu.VMEM((NPN,), jnp.float32),
        pltpu.VMEM((NPN,), jnp.float32),
        pltpu.VMEM((NPN,), jnp.float32),
        pltpu.VMEM((NPN,), jnp.float32),
        pltpu.VMEM((NPN,), jnp.float32),
        pltpu.VMEM((NPN,), jnp.float32),
        pltpu.VMEM((NPN,), jnp.float32),
        pltpu.VMEM((NPN,), jnp.float32),
        pltpu.VMEM((NPN,), jnp.float32),
        pltpu.VMEM((NPN,), jnp.float32),
        pltpu.VMEM((NPN,), jnp.float32),
        pltpu.VMEM((NPN,), jnp.float32),
        pltpu.VMEM((NPN,), jnp.float32),
        pltpu.VMEM((NPN,), jnp.float32),
        pltpu.VMEM((NPN,), jnp.float32),
        pltpu.VMEM((NPN,), jnp.float32),
    ])
def _sc_tables(x_hbm, y_hbm, id_hbm, st_hbm, par_hbm, out_hbm,
               xb0, xb1, yb0, yb1, ib0, ib1, sem0, sem1, svec, pvec,
               ra0, ra1, ra2, ra3, rb0, rb1, rb2, rb3,
               sa0, sa1, sa2, sa3, sb0, sb1, sb2, sb3):
    c = lax.axis_index("c")
    s = lax.axis_index("s")
    wid = s * 2 + c

    pltpu.sync_copy(st_hbm, svec)
    pltpu.sync_copy(par_hbm, pvec)
    invg = pvec[...]

    iota = lax.iota(jnp.int32, 16)
    ip = jnp.maximum(iota - 1, 0)
    inx = jnp.minimum(iota + 1, 15)
    rev = 15 - iota

    v0 = svec[pl.ds(0, 16)]
    v1 = svec[pl.ds(16, 16)]
    v2 = svec[pl.ds(32, 16)]

    def pick(w):
        z = (jnp.where(iota == w, v0, 0) + jnp.where(iota == w - 16, v1, 0)
             + jnp.where(iota == w - 32, v2, 0))
        return jnp.sum(z.astype(jnp.float32)).astype(jnp.int32)

    p_lo = pick(wid)
    p_hi = pick(wid + 1)
    net_lo = wid * NPN
    p0 = p_lo & jnp.int32(-8)
    nch = (p_hi - p0 + (CHUNK - 1)) // CHUNK

    zf = jnp.zeros((16,), jnp.float32)
    lo = jnp.full((16,), FLO, jnp.float32)
    hi16 = jnp.full((16,), FHI, jnp.float32)

    def zbody(i, carry):
        sl = pl.ds(i * 16, 16)
        for t in (ra0, ra2, rb0, rb2):
            t[sl] = lo
        for t in (ra1, ra3, rb1, rb3):
            t[sl] = hi16
        for t in (sa0, sa1, sa2, sa3, sb0, sb1, sb2, sb3):
            t[sl] = zf
        return carry
    lax.fori_loop(0, NPN // 16, zbody, 0)

    slots = ((xb0, yb0, ib0, sem0), (xb1, yb1, ib1, sem1))

    def off_of(k):
        return pl.multiple_of(p0 + k * CHUNK, 8)

    def start(slot, off):
        xb, yb, ib, sem = slots[slot]
        pltpu.async_copy(x_hbm.at[pl.ds(off, CHUNK)], xb, sem)
        pltpu.async_copy(y_hbm.at[pl.ds(off, CHUNK)], yb, sem)
        pltpu.async_copy(id_hbm.at[pl.ds(off, CHUNK)], ib, sem)

    def wait(slot):
        xb, yb, ib, sem = slots[slot]
        pltpu.make_async_copy(x_hbm.at[pl.ds(0, CHUNK)], xb, sem).wait()
        pltpu.make_async_copy(y_hbm.at[pl.ds(0, CHUNK)], yb, sem).wait()
        pltpu.make_async_copy(id_hbm.at[pl.ds(0, CHUNK)], ib, sem).wait()

    rsets = ((ra0, ra1, ra2, ra3), (rb0, rb1, rb2, rb3))
    ssets = ((sa0, sa1, sa2, sa3), (sb0, sb1, sb2, sb3))

    def vreg_f(xbuf, ybuf, idbuf, j, tset):
        sl = pl.ds(j * 16, 16)
        ids = idbuf[sl]
        xv = xbuf[sl]
        yv = ybuf[sl]
        relu = plsc.bitcast(ids - net_lo, jnp.uint32)
        inb = relu < NPN
        idx = plsc.bitcast(jnp.minimum(relu, NPN - 1), jnp.int32)
        tail = (iota == 15) | (ids != _vshift(ids, inx))
        head = ids != _vshift(ids, ip)
        tmask = tail & inb
        h = plsc.cummax(jnp.where(head, iota, 0))
        hm1 = jnp.maximum(h - 1, 0)
        hz = h == 0
        tl = jnp.where(tail, 1, 0)
        g = plsc.cummax(jnp.where(_vshift(tl, rev) != 0, iota, 0))
        nt = _vshift(15 - g, rev)
        hi = jnp.left_shift(relu, 19)
        bx = jnp.right_shift(plsc.bitcast(xv, jnp.uint32), 12)
        by = jnp.right_shift(plsc.bitcast(yv, jnp.uint32), 12)
        xg = xv * invg
        yg = yv * invg
        rs = rsets[tset]
        ss = ssets[tset]
        for rt, st, b, neg, vg in ((rs[0], ss[0], bx, False, xg),
                                   (rs[1], ss[1], MASK19 - bx, True, xg),
                                   (rs[2], ss[2], by, False, yg),
                                   (rs[3], ss[3], MASK19 - by, True, yg)):
            comb = jnp.where(inb, hi | b, jnp.uint32(0))
            cm = plsc.bitcast(plsc.cummax(comb), jnp.int32)
            q = _vshift(cm, nt) & MASK19
            b2 = (MASK19 - q) if neg else q
            rrun = plsc.bitcast(jnp.left_shift(b2, 12), jnp.float32) * invg
            rold = plsc.load_gather(rt, [idx])
            rnew = jnp.minimum(rold, rrun) if neg else jnp.maximum(rold, rrun)
            arg = (rnew - vg) if neg else (vg - rnew)
            e = jnp.where(inb, jnp.exp(arg), 0.0)
            cs = jnp.cumsum(e)
            pfx = jnp.where(hz, 0.0, _vshift(cs, hm1))
            sold = plsc.load_gather(st, [idx])
            dl = (rnew - rold) if neg else (rold - rnew)
            snew = sold * jnp.exp(dl) + (cs - pfx)
            plsc.store_scatter(st, [idx], snew, mask=tmask)
            plsc.store_scatter(rt, [idx], rnew, mask=tmask)

    start(0, off_of(0))

    def pair_body(p, carry):
        k0 = 2 * p
        start(1, off_of(k0 + 1))
        wait(0)

        def vb0(j, cc):
            vreg_f(slots[0][0], slots[0][1], slots[0][2], 2 * j, 0)
            vreg_f(slots[0][0], slots[0][1], slots[0][2], 2 * j + 1, 1)
            return cc
        lax.fori_loop(0, CHUNK // 32, vb0, 0)
        start(0, off_of(k0 + 2))
        wait(1)

        def vb1(j, cc):
            vreg_f(slots[1][0], slots[1][1], slots[1][2], 2 * j, 0)
            vreg_f(slots[1][0], slots[1][1], slots[1][2], 2 * j + 1, 1)
            return cc
        lax.fori_loop(0, CHUNK // 32, vb1, 0)
        return carry
    lax.fori_loop(0, (nch + 1) // 2, pair_body, 0)
    wait(0)

    base = wid * NPN
    for row, t in enumerate((sa0, sa1, sa2, sa3, sb0, sb1, sb2, sb3,
                             ra0, ra1, ra2, ra3, rb0, rb1, rb2, rb3)):
        pltpu.sync_copy(
            t, out_hbm.at[pl.ds(pl.multiple_of(row * NPAD + base, 8), NPN)])


def _tc_final(tab_ref, mask_ref, g_ref, out_ref):
    g = g_ref[0, 0]
    m = mask_ref[...]
    one = jnp.float32(1.0)

    def merge(sa, sb, ra, rb, neg):
        r = jnp.minimum(ra, rb) if neg else jnp.maximum(ra, rb)
        da = (r - ra) if neg else (ra - r)
        db = (r - rb) if neg else (rb - r)
        s = sa * jnp.exp(da) + sb * jnp.exp(db)
        return r, s

    rp_x, sp_x = merge(tab_ref[0], tab_ref[4], tab_ref[8], tab_ref[12], False)
    rm_x, sm_x = merge(tab_ref[1], tab_ref[5], tab_ref[9], tab_ref[13], True)
    rp_y, sp_y = merge(tab_ref[2], tab_ref[6], tab_ref[10], tab_ref[14], False)
    rm_y, sm_y = merge(tab_ref[3], tab_ref[7], tab_ref[11], tab_ref[15], True)

    valid = (sp_x > 0.0) & (m > 0.0)

    def lg(sv):
        return jnp.log(jnp.where(valid, sv, one))

    tot = jnp.where(
        valid,
        (rp_x - rm_x) + (rp_y - rm_y)
        + lg(sp_x) + lg(sm_x) + lg(sp_y) + lg(sm_y),
        0.0)
    out_ref[0, 0] = g * jnp.sum(tot)


def kernel(pos, pin2net_map, net_mask, gamma):
    x = pos[:NPIN]
    y = pos[NPIN:]
    bounds = jnp.arange(0, (NSUB + 1) * NPN, NPN, dtype=jnp.int32)
    starts = jnp.searchsorted(pin2net_map, bounds).astype(jnp.int32)
    st = jnp.zeros((48,), jnp.int32).at[:33].set(starts)
    xp = jnp.concatenate([x, jnp.zeros((PAD,), jnp.float32)])
    yp = jnp.concatenate([y, jnp.zeros((PAD,), jnp.float32)])
    idp = jnp.concatenate([pin2net_map,
                           jnp.full((PAD,), SENT, jnp.int32)])
    g = jnp.asarray(gamma, jnp.float32)
    par = jnp.full((16,), 1.0, jnp.float32) / g

    tabs = _sc_tables(xp, yp, idp, st, par)

    mask = jnp.concatenate([net_mask.astype(jnp.float32),
                            jnp.zeros((NPAD - NNET,), jnp.float32)])
    res = pl.pallas_call(
        _tc_final,
        out_shape=jax.ShapeDtypeStruct((1, 1), jnp.float32),
        in_specs=[
            pl.BlockSpec(memory_space=pltpu.VMEM),
            pl.BlockSpec(memory_space=pltpu.VMEM),
            pl.BlockSpec(memory_space=pltpu.SMEM),
        ],
        out_specs=pl.BlockSpec(memory_space=pltpu.SMEM),
    )(tabs.reshape(16, 800, 128), mask.reshape(800, 128), g.reshape(1, 1))
    return res[0, 0]
